# Initial kernel scaffold; baseline (speedup 1.0000x reference)
import jax
import jax.numpy as jnp
from jax.experimental import pallas as pl


def kernel(x, edge_index, num_nodes, W, W0):
    raise NotImplementedError("write your pallas kernel here")


# speedup vs baseline: 2.5518x; 2.5518x over previous
import functools

import jax
import jax.numpy as jnp
from jax import lax
from jax.experimental import pallas as pl
from jax.experimental.pallas import tpu as pltpu
from jax.experimental.pallas import tpu_sc as plsc

NC = 2
NS = 16
NW = NC * NS
K = 128
NCH = 40
E_PAD = NW * NCH * K
ZROWS = 64


def _matmul_body(x_ref, w_ref, xa_ref, xb_ref):
    p = jnp.dot(x_ref[...], w_ref[...], preferred_element_type=jnp.float32)
    xa_ref[...] = p[:, :128]
    xb_ref[...] = p[:, 128:]


def _combine_body(x_ref, w0_ref, aggA_ref, aggB_ref, deg_ref, out_ref):
    out0 = jnp.dot(x_ref[...], w0_ref[...], preferred_element_type=jnp.float32)
    a = aggA_ref[0] + aggA_ref[1]
    b = aggB_ref[0] + aggB_ref[1]
    d = jnp.sum(deg_ref[...], axis=0)[:, None]
    r = 1.0 / jnp.maximum(d, 1.0)
    out_ref[...] = (out0 + jnp.concatenate([a, b], axis=1)) * r


def _sc_body(xa, xb, srcb, dstb, aggA, aggB, degout,
             src_v, dst_v, rows_v, deg_v, zeros_v, acc, sem):
    c = lax.axis_index("c")
    s = lax.axis_index("s")
    w = c * NS + s
    n_acc = acc.shape[0]
    rows_per_tile = n_acc // NS
    base = s * rows_per_tile

    pltpu.sync_copy(srcb.at[w], src_v)
    pltpu.sync_copy(dstb.at[w], dst_v)

    def _init_zeros(i, _):
        for cc in range(8):
            zeros_v[i, pl.ds(cc * 16, 16)] = jnp.zeros((16,), jnp.float32)
        return _
    lax.fori_loop(0, ZROWS, _init_zeros, 0)

    def _init_deg(i, _):
        deg_v[pl.ds(i * 16, 16)] = jnp.zeros((16,), jnp.float32)
        return _
    lax.fori_loop(0, n_acc // 16, _init_deg, 0)

    ones16 = jnp.ones((16,), jnp.float32)
    for p in range(2):
        xw = xa if p == 0 else xb
        agg = aggA if p == 0 else aggB

        for z in range(rows_per_tile // ZROWS):
            pltpu.sync_copy(zeros_v, acc.at[pl.ds(base + z * ZROWS, ZROWS)])
        plsc.subcore_barrier()

        if p == 0:
            def _chunk(j, _):
                pltpu.async_copy(xw.at[src_v.at[j]], rows_v, sem).wait()
                pltpu.sync_copy(rows_v, acc.at[dst_v.at[j]], add=True)
                for v in range(K // 16):
                    idx = dst_v[j, pl.ds(v * 16, 16)]
                    plsc.addupdate_scatter(deg_v, [idx], ones16)
                return _
        else:
            def _chunk(j, _):
                pltpu.async_copy(xw.at[src_v.at[j]], rows_v, sem).wait()
                pltpu.sync_copy(rows_v, acc.at[dst_v.at[j]], add=True)
                return _
        lax.fori_loop(0, NCH, _chunk, 0)
        plsc.subcore_barrier()

        pltpu.sync_copy(acc.at[pl.ds(base, rows_per_tile)],
                        agg.at[c].at[pl.ds(base, rows_per_tile)])
        plsc.subcore_barrier()

    pltpu.sync_copy(deg_v, degout.at[w])


def kernel(x, edge_index, num_nodes, W, W0):
    n = x.shape[0]
    d_in = x.shape[1]
    d_out = W.shape[1]
    e = edge_index.shape[1]
    rows_per_tile = -(-(n + 1) // (NS * ZROWS)) * ZROWS
    n_acc = rows_per_tile * NS

    src = edge_index[0]
    dst = edge_index[1]
    pad = E_PAD - e
    src_b = jnp.concatenate(
        [src, jnp.zeros((pad,), jnp.int32)]).reshape(NW, NCH, K)
    dst_b = jnp.concatenate(
        [dst, jnp.full((pad,), n, jnp.int32)]).reshape(NW, NCH, K)

    bm = 1000
    xa, xb = pl.pallas_call(
        _matmul_body,
        grid=(n // bm,),
        in_specs=[
            pl.BlockSpec((bm, d_in), lambda i: (i, 0)),
            pl.BlockSpec((d_in, d_out), lambda i: (0, 0)),
        ],
        out_specs=[
            pl.BlockSpec((bm, 128), lambda i: (i, 0)),
            pl.BlockSpec((bm, 128), lambda i: (i, 0)),
        ],
        out_shape=[
            jax.ShapeDtypeStruct((n, 128), jnp.float32),
            jax.ShapeDtypeStruct((n, 128), jnp.float32),
        ],
    )(x, W)

    mesh = plsc.VectorSubcoreMesh(core_axis_name="c", subcore_axis_name="s")
    sc_call = pl.kernel(
        _sc_body,
        out_type=[
            jax.ShapeDtypeStruct((NC, n_acc, 128), jnp.float32),
            jax.ShapeDtypeStruct((NC, n_acc, 128), jnp.float32),
            jax.ShapeDtypeStruct((NW, n_acc), jnp.float32),
        ],
        mesh=mesh,
        scratch_types=[
            pltpu.VMEM((NCH, K), jnp.int32),
            pltpu.VMEM((NCH, K), jnp.int32),
            pltpu.VMEM((K, 128), jnp.float32),
            pltpu.VMEM((n_acc,), jnp.float32),
            pltpu.VMEM((ZROWS, 128), jnp.float32),
            pltpu.VMEM_SHARED((n_acc, 128), jnp.float32),
            pltpu.SemaphoreType.DMA,
        ],
        compiler_params=pltpu.CompilerParams(needs_layout_passes=False),
    )
    aggA, aggB, deg = sc_call(xa, xb, src_b, dst_b)

    bc = 1024
    gc = -(-n // bc)
    out = pl.pallas_call(
        _combine_body,
        grid=(gc,),
        in_specs=[
            pl.BlockSpec((bc, d_in), lambda i: (i, 0)),
            pl.BlockSpec((d_in, d_out), lambda i: (0, 0)),
            pl.BlockSpec((NC, bc, 128), lambda i: (0, i, 0)),
            pl.BlockSpec((NC, bc, 128), lambda i: (0, i, 0)),
            pl.BlockSpec((NW, bc), lambda i: (0, i)),
        ],
        out_specs=pl.BlockSpec((bc, d_out), lambda i: (i, 0)),
        out_shape=jax.ShapeDtypeStruct((n, d_out), jnp.float32),
    )(x, W0, aggA, aggB, deg)
    return out
